# Initial kernel scaffold; baseline (speedup 1.0000x reference)
import jax
import jax.numpy as jnp
from jax.experimental import pallas as pl


def kernel(ids, gen_embed):
    raise NotImplementedError("write your pallas kernel here")


# speedup vs baseline: 5.3271x; 5.3271x over previous
import functools

import jax
import jax.numpy as jnp
from jax import lax
from jax.experimental import pallas as pl
from jax.experimental.pallas import tpu as pltpu
from jax.experimental.pallas import tpu_sc as plsc

_D = 32
_B = 4096 * 200
_NC, _NS = 2, 16
_NW = _NC * _NS
_BPW = _B // _NW
_C = 1600
_NCHUNK = _BPW // _C

_mesh = plsc.VectorSubcoreMesh(core_axis_name="c", subcore_axis_name="s")


@functools.partial(
    pl.kernel,
    mesh=_mesh,
    out_type=jax.ShapeDtypeStruct((_B, _D), jnp.float32),
    scratch_types=[
        pltpu.VMEM((_C,), jnp.int32),
        pltpu.VMEM((_C, _D), jnp.float32),
        pltpu.SemaphoreType.DMA,
    ],
    compiler_params=pltpu.CompilerParams(use_tc_tiling_on_sc=False),
)
def _gather_kernel(ids_hbm, table_hbm, out_hbm, idx_v, rows_v, sem):
    wid = lax.axis_index("s") * _NC + lax.axis_index("c")
    base = wid * _BPW

    def body(i, carry):
        off = base + i * _C
        pltpu.sync_copy(ids_hbm.at[pl.ds(off, _C)], idx_v)
        pltpu.async_copy(table_hbm.at[idx_v], rows_v, sem).wait()
        pltpu.sync_copy(rows_v, out_hbm.at[pl.ds(off, _C)])
        return carry

    lax.fori_loop(0, _NCHUNK, body, 0)


def kernel(ids, gen_embed):
    flat = ids.reshape(_B)
    out = _gather_kernel(flat, gen_embed)
    return out.reshape(ids.shape[0], ids.shape[1], _D)


# speedup vs baseline: 5.4621x; 1.0254x over previous
import functools

import jax
import jax.numpy as jnp
from jax import lax
from jax.experimental import pallas as pl
from jax.experimental.pallas import tpu as pltpu
from jax.experimental.pallas import tpu_sc as plsc

_D = 32
_B = 4096 * 200
_NC, _NS = 2, 16
_NW = _NC * _NS
_BPW = _B // _NW
_C = 640
_NCHUNK = _BPW // _C
_K = 4
_LAG = 2

_mesh = plsc.VectorSubcoreMesh(core_axis_name="c", subcore_axis_name="s")


@functools.partial(
    pl.kernel,
    mesh=_mesh,
    out_type=jax.ShapeDtypeStruct((_B, _D), jnp.float32),
    scratch_types=(
        [pltpu.VMEM((_BPW,), jnp.int32)]
        + [pltpu.VMEM((_C, _D), jnp.float32) for _ in range(_K)]
        + [pltpu.SemaphoreType.DMA for _ in range(2 * _K)]
    ),
    compiler_params=pltpu.CompilerParams(use_tc_tiling_on_sc=False),
)
def _gather_kernel(ids_hbm, table_hbm, out_hbm, idx_v, *bufs_and_sems):
    rows = bufs_and_sems[:_K]
    gsem = bufs_and_sems[_K:2 * _K]
    osem = bufs_and_sems[2 * _K:]

    wid = lax.axis_index("s") * _NC + lax.axis_index("c")
    base = wid * _BPW

    pltpu.sync_copy(ids_hbm.at[pl.ds(base, _BPW)], idx_v)

    def start_gather(i):
        b = i % _K
        pltpu.async_copy(
            table_hbm.at[idx_v.at[pl.ds(i * _C, _C)]], rows[b], gsem[b])

    def drain_to_out(i):
        b = i % _K
        pltpu.make_async_copy(
            table_hbm.at[idx_v.at[pl.ds(i * _C, _C)]], rows[b], gsem[b]).wait()
        pltpu.async_copy(rows[b], out_hbm.at[pl.ds(base + i * _C, _C)], osem[b])

    def wait_out(i):
        b = i % _K
        pltpu.make_async_copy(
            rows[b], out_hbm.at[pl.ds(base + i * _C, _C)], osem[b]).wait()

    for i in range(_NCHUNK):
        if i >= _K:
            wait_out(i - _K)
        start_gather(i)
        if i >= _LAG:
            drain_to_out(i - _LAG)
    for i in range(_NCHUNK - _LAG, _NCHUNK):
        drain_to_out(i)
    for i in range(_NCHUNK - _K, _NCHUNK):
        wait_out(i)


def kernel(ids, gen_embed):
    flat = ids.reshape(_B)
    out = _gather_kernel(flat, gen_embed)
    return out.reshape(ids.shape[0], ids.shape[1], _D)
